# baseline (device time: 13718 ns/iter reference)
import jax
import jax.numpy as jnp
from jax import lax
from jax.experimental import pallas as pl
from jax.experimental.pallas import tpu as pltpu

N_DEV = 8
_MASK_ORDER = (7, 3, 5, 6, 1, 2, 4)


def _coords(l):
    z = l // 4
    p = l % 4
    y = p // 2
    a = p % 2
    x = a + y - 2 * a * y
    return x, y, z


def _logical(x, y, z):
    s = x + y - 2 * x * y
    return 4 * z + 2 * y + s


def kernel(x, w_mat):
    m_per, k = x.shape
    n = w_mat.shape[1]
    n_per = n // N_DEV

    def body(x_ref, w_ref, out_ref, bsend, brecv,
             send_sems, recv_sems, ready_sems):
        my_i = lax.axis_index("i")
        mx, my, mz = _coords(my_i)

        barrier_sem = pltpu.get_barrier_semaphore()
        pl.semaphore_signal(barrier_sem, inc=1)
        pl.semaphore_wait(barrier_sem, 1)

        partners = []
        for mask in _MASK_ORDER:
            dx, dy, dz = mask & 1, (mask >> 1) & 1, mask >> 2
            partners.append(_logical(mx + dx - 2 * mx * dx,
                                     my + dy - 2 * my * dy,
                                     mz + dz - 2 * mz * dz))

        for r, dst in enumerate(partners):
            pl.semaphore_signal(
                ready_sems.at[r], inc=1,
                device_id=(dst,), device_id_type=pl.DeviceIdType.MESH,
            )

        def drain(r):
            recv = pltpu.make_async_remote_copy(
                src_ref=bsend.at[r],
                dst_ref=brecv.at[r],
                send_sem=send_sems.at[r],
                recv_sem=recv_sems.at[r],
                device_id=(partners[r],),
                device_id_type=pl.DeviceIdType.MESH,
            )
            recv.wait_recv()
            out_ref[pl.ds(partners[r] * m_per, m_per), :] = (
                brecv[r].astype(jnp.float32)
            )

        _DRAIN_LAG = 4

        xv = x_ref[:, :]
        rdmas = []
        for r, dst in enumerate(partners):
            y = jnp.maximum(
                jnp.dot(xv, w_ref[:, pl.ds(dst * n_per, n_per)],
                        preferred_element_type=jnp.float32),
                0.0,
            )
            bsend[r] = y.astype(jnp.bfloat16)
            pl.semaphore_wait(ready_sems.at[r], 1)
            rdma = pltpu.make_async_remote_copy(
                src_ref=bsend.at[r],
                dst_ref=brecv.at[r],
                send_sem=send_sems.at[r],
                recv_sem=recv_sems.at[r],
                device_id=(dst,),
                device_id_type=pl.DeviceIdType.MESH,
            )
            rdma.start()
            rdmas.append(rdma)
            if r >= _DRAIN_LAG:
                drain(r - _DRAIN_LAG)

        out_ref[pl.ds(my_i * m_per, m_per), :] = jnp.maximum(
            jnp.dot(xv, w_ref[:, pl.ds(my_i * n_per, n_per)],
                    preferred_element_type=jnp.float32),
            0.0,
        )

        for r in range(N_DEV - 1 - _DRAIN_LAG, N_DEV - 1):
            drain(r)

        for rdma in rdmas:
            rdma.wait_send()

    return pl.pallas_call(
        body,
        out_shape=jax.ShapeDtypeStruct((N_DEV * m_per, n_per), jnp.float32),
        in_specs=[
            pl.BlockSpec(memory_space=pltpu.VMEM),
            pl.BlockSpec(memory_space=pltpu.VMEM),
        ],
        out_specs=pl.BlockSpec(memory_space=pltpu.VMEM),
        scratch_shapes=[
            pltpu.VMEM((N_DEV - 1, m_per, n_per), jnp.bfloat16),
            pltpu.VMEM((N_DEV - 1, m_per, n_per), jnp.bfloat16),
            pltpu.SemaphoreType.DMA((N_DEV - 1,)),
            pltpu.SemaphoreType.DMA((N_DEV - 1,)),
            pltpu.SemaphoreType.REGULAR((N_DEV - 1,)),
        ],
        compiler_params=pltpu.CompilerParams(collective_id=0),
    )(x, w_mat)


# device time: 12315 ns/iter; 1.1139x vs baseline; 1.1139x over previous
import jax
import jax.numpy as jnp
from jax import lax
from jax.experimental import pallas as pl
from jax.experimental.pallas import tpu as pltpu

N_DEV = 8
_MASK_ORDER = (7, 3, 5, 6, 1, 2, 4)


def _coords(l):
    z = l // 4
    p = l % 4
    y = p // 2
    a = p % 2
    x = a + y - 2 * a * y
    return x, y, z


def _logical(x, y, z):
    s = x + y - 2 * x * y
    return 4 * z + 2 * y + s


def kernel(x, w_mat):
    m_per, k = x.shape
    n = w_mat.shape[1]
    n_per = n // N_DEV

    def body(x_ref, w_ref, out_ref, bsend, brecv,
             send_sems, recv_sems, ready_sems):
        my_i = lax.axis_index("i")
        mx, my, mz = _coords(my_i)

        barrier_sem = pltpu.get_barrier_semaphore()
        pl.semaphore_signal(barrier_sem, inc=1)
        pl.semaphore_wait(barrier_sem, 1)

        partners = []
        for mask in _MASK_ORDER:
            dx, dy, dz = mask & 1, (mask >> 1) & 1, mask >> 2
            partners.append(_logical(mx + dx - 2 * mx * dx,
                                     my + dy - 2 * my * dy,
                                     mz + dz - 2 * mz * dz))

        for r, dst in enumerate(partners):
            pl.semaphore_signal(
                ready_sems.at[r], inc=1,
                device_id=(dst,), device_id_type=pl.DeviceIdType.MESH,
            )

        def drain(r):
            recv = pltpu.make_async_remote_copy(
                src_ref=bsend.at[r],
                dst_ref=brecv.at[r],
                send_sem=send_sems.at[r],
                recv_sem=recv_sems.at[r],
                device_id=(partners[r],),
                device_id_type=pl.DeviceIdType.MESH,
            )
            recv.wait_recv()
            out_ref[pl.ds(partners[r] * m_per, m_per), :] = (
                brecv[r].astype(jnp.float32)
            )

        xv = x_ref[:, :]
        rdmas = []
        for r, dst in enumerate(partners):
            y = jnp.maximum(
                jnp.dot(xv, w_ref[:, pl.ds(dst * n_per, n_per)],
                        preferred_element_type=jnp.float32),
                0.0,
            )
            bsend[r] = y.astype(jnp.bfloat16)
            pl.semaphore_wait(ready_sems.at[r], 1)
            rdma = pltpu.make_async_remote_copy(
                src_ref=bsend.at[r],
                dst_ref=brecv.at[r],
                send_sem=send_sems.at[r],
                recv_sem=recv_sems.at[r],
                device_id=(dst,),
                device_id_type=pl.DeviceIdType.MESH,
            )
            rdma.start()
            rdmas.append(rdma)

        out_ref[pl.ds(my_i * m_per, m_per), :] = jnp.maximum(
            jnp.dot(xv, w_ref[:, pl.ds(my_i * n_per, n_per)],
                    preferred_element_type=jnp.float32),
            0.0,
        )

        for r in range(N_DEV - 1):
            drain(r)

        for rdma in rdmas:
            rdma.wait_send()

    return pl.pallas_call(
        body,
        out_shape=jax.ShapeDtypeStruct((N_DEV * m_per, n_per), jnp.float32),
        in_specs=[
            pl.BlockSpec(memory_space=pltpu.VMEM),
            pl.BlockSpec(memory_space=pltpu.VMEM),
        ],
        out_specs=pl.BlockSpec(memory_space=pltpu.VMEM),
        scratch_shapes=[
            pltpu.VMEM((N_DEV - 1, m_per, n_per), jnp.bfloat16),
            pltpu.VMEM((N_DEV - 1, m_per, n_per), jnp.bfloat16),
            pltpu.SemaphoreType.DMA((N_DEV - 1,)),
            pltpu.SemaphoreType.DMA((N_DEV - 1,)),
            pltpu.SemaphoreType.REGULAR((N_DEV - 1,)),
        ],
        compiler_params=pltpu.CompilerParams(collective_id=0),
    )(x, w_mat)
